# baseline (speedup 1.0000x reference)
import functools

import jax
import jax.numpy as jnp
from jax import lax
from jax.experimental import pallas as pl
from jax.experimental.pallas import tpu as pltpu
from jax.experimental.pallas import tpu_sc as plsc

NU = 50000
D = 64
H = 32
NE = 800000
B = 4096
NC = 2
NS = 16
W = 256
RW = W // 128
WIN = 396
EPT = W * WIN
NE_PAD = EPT * 8
NE2 = 2 * NE_PAD
NROWS = NE2 // 128
ACCR = 50048
ZR = 32
DR = 128


def _propagate(src2d, dst2d, wcat, ut2, it2, twpad):
    mesh = plsc.VectorSubcoreMesh(core_axis_name="c", subcore_axis_name="s")
    otype = [jax.ShapeDtypeStruct((2 * NU, H), jnp.float32)] * 4

    ibuf_types = []
    for _ in range(6):
        ibuf_types += [
            pltpu.VMEM((RW, 128), jnp.int32),
            pltpu.VMEM((RW, 128), jnp.int32),
            pltpu.VMEM((W,), jnp.float32),
            pltpu.SemaphoreType.DMA,
        ]

    rbuf_types = []
    for _ in range(3):
        rbuf_types += [
            pltpu.VMEM((W, H), jnp.float32),
            pltpu.SemaphoreType.DMA,
            pltpu.SemaphoreType.DMA,
        ]

    @functools.partial(
        pl.kernel,
        out_type=otype,
        mesh=mesh,
        compiler_params=pltpu.CompilerParams(needs_layout_passes=False,
                                             use_tc_tiling_on_sc=False),
        scratch_types=ibuf_types + rbuf_types + [
            pltpu.VMEM_SHARED((ACCR, H), jnp.float32),
            pltpu.VMEM((16,), jnp.float32),
            pltpu.VMEM((ZR, H), jnp.float32),
        ],
    )
    def k1(src_hbm, dst_hbm, w_hbm, ut_hbm, it_hbm, tw_hbm,
           u1, i1, u2, i2,
           sb0, gb0, wb0, is0, sb1, gb1, wb1, is1,
           sb2, gb2, wb2, is2, sb3, gb3, wb3, is3,
           sb4, gb4, wb4, is4, sb5, gb5, wb5, is5,
           row0, gs0, ss0, row1, gs1, ss1, row2, gs2, ss2,
           acc, twv, zbuf):
        c = lax.axis_index("c")
        s = lax.axis_index("s")
        ins = ((sb0, gb0, wb0, is0), (sb1, gb1, wb1, is1),
               (sb2, gb2, wb2, is2), (sb3, gb3, wb3, is3),
               (sb4, gb4, wb4, is4), (sb5, gb5, wb5, is5))
        rows = ((row0, gs0, ss0), (row1, gs1, ss1), (row2, gs2, ss2))

        pltpu.sync_copy(tw_hbm, twv)
        x = twv[...]
        e = jnp.exp(x - jnp.max(x))
        tw_all = e / jnp.broadcast_to(jnp.sum(e), (16,))
        my_tw = jnp.where(s < 8, tw_all[0], tw_all[1])

        @pl.loop(0, ZR)
        def _zb(r):
            zero16 = jnp.zeros((16,), jnp.float32)
            zbuf[r, pl.ds(0, 16)] = zero16
            zbuf[r, pl.ds(16, 16)] = zero16

        def do_pass(gather_tbl, interleaved, scat_is_src, out_tbl,
                    first=False):
            sidx_hbm = src_hbm if scat_is_src else dst_hbm
            gath_hbm = dst_hbm if scat_is_src else src_hbm

            if first:
                @pl.loop(s, ACCR // ZR, step=NS)
                def _z(m):
                    pltpu.sync_copy(zbuf, acc.at[pl.ds(m * ZR, ZR), :])

            def in_copies(nwin, bi):
                sb, gb, wb, isem = ins[bi]
                rowbase = s * (EPT // 128) + nwin * RW
                return (
                    pltpu.make_async_copy(
                        sidx_hbm.at[pl.ds(rowbase, RW), :], sb, isem),
                    pltpu.make_async_copy(
                        gath_hbm.at[pl.ds(rowbase, RW), :], gb, isem),
                    pltpu.make_async_copy(
                        w_hbm.at[pl.ds(rowbase * 128, W)], wb, isem),
                )

            def issue_in(nwin, bi):
                for cp in in_copies(nwin, bi):
                    cp.start()

            def wait_in(nwin, bi):
                for cp in in_copies(nwin, bi):
                    cp.wait()

            def transform(bi):
                _, gb, wb, _ = ins[bi]

                @pl.loop(0, RW)
                def _t(j):
                    for kk in range(8):
                        sl = pl.ds(kk * 16, 16)
                        v = gb[j, sl]
                        if interleaved:
                            gb[j, sl] = v * 2 + c
                        else:
                            gb[j, sl] = v + c * NU

            def gather_copies(bi, ri):
                _, gb, _, _ = ins[bi]
                row, gsem, _ = rows[ri]
                return [
                    pltpu.make_async_copy(
                        gather_tbl.at[gb.at[j]],
                        row.at[pl.ds(j * 128, 128), :], gsem)
                    for j in range(RW)
                ]

            def start_gather(bi, ri):
                for cp in gather_copies(bi, ri):
                    cp.start()

            def wait_gather(bi, ri):
                for cp in gather_copies(bi, ri):
                    cp.wait()

            def multiply(bi, ri):
                _, _, wb, _ = ins[bi]
                row, _, _ = rows[ri]

                @plsc.parallel_loop(0, W // 16)
                def _m(g):
                    w16 = wb[pl.ds(g * 16, 16)] * my_tw
                    for l in range(16):
                        wv = jnp.broadcast_to(w16[l], (16,))
                        ee = g * 16 + l
                        row[ee, pl.ds(0, 16)] = row[ee, pl.ds(0, 16)] * wv
                        row[ee, pl.ds(16, 16)] = row[ee, pl.ds(16, 16)] * wv

            def start_scatter(bi, ri):
                sb, _, _, _ = ins[bi]
                row, _, ssem = rows[ri]
                for j in range(RW):
                    pltpu.async_copy(
                        row.at[pl.ds(j * 128, 128), :],
                        acc.at[sb.at[j]], ssem, add=True)

            def wait_scatter(bi, ri):
                sb, _, _, _ = ins[bi]
                row, _, ssem = rows[ri]
                for j in range(RW):
                    pltpu.make_async_copy(
                        row.at[pl.ds(j * 128, 128), :],
                        acc.at[sb.at[j]], ssem).wait()

            issue_in(0, 0)
            issue_in(1, 1)
            issue_in(2, 2)
            plsc.subcore_barrier()

            NQ = WIN // 6

            @pl.loop(0, NQ)
            def _six(q):
                v0 = 6 * q
                for k in range(6):
                    v = v0 + k
                    ik3 = (k + 3) % 6
                    ik4 = (k + 4) % 6
                    r = k % 3
                    r1 = (k + 1) % 3
                    wait_in(v, k)
                    transform(k)
                    if k <= 2:
                        @pl.when(q > 0)
                        def _ws(ik3=ik3, r=r):
                            wait_scatter(ik3, r)

                        issue_in(v + 3, ik3)
                    else:
                        wait_scatter(ik3, r)

                        @pl.when(q < NQ - 1)
                        def _ii(v=v, ik3=ik3):
                            issue_in(v + 3, ik3)

                    start_gather(k, r)

                    def _tail(ik4=ik4, r1=r1):
                        wait_gather(ik4, r1)
                        multiply(ik4, r1)
                        start_scatter(ik4, r1)

                    if k <= 1:
                        pl.when(q > 0)(_tail)
                    else:
                        _tail()

            wait_gather(4, 1)
            multiply(4, 1)
            start_scatter(4, 1)
            wait_gather(5, 2)
            multiply(5, 2)
            start_scatter(5, 2)
            wait_scatter(3, 0)
            wait_scatter(4, 1)
            wait_scatter(5, 2)

            plsc.subcore_barrier()

            @pl.loop(s, NU // DR, step=NS)
            def _d(m):
                r0 = m * DR
                pltpu.sync_copy(
                    acc.at[pl.ds(r0, DR), :],
                    out_tbl.at[pl.ds(c * NU + r0, DR), :],
                )
                for kz in range(DR // ZR):
                    pltpu.sync_copy(
                        zbuf, acc.at[pl.ds(r0 + kz * ZR, ZR), :])

            @pl.when(s == (NU // DR) % NS)
            def _tail():
                r0 = (NU // DR) * DR
                pltpu.sync_copy(
                    acc.at[pl.ds(r0, NU - r0), :],
                    out_tbl.at[pl.ds(c * NU + r0, NU - r0), :],
                )
                for kz in range((NU - r0) // ZR):
                    pltpu.sync_copy(
                        zbuf, acc.at[pl.ds(r0 + kz * ZR, ZR), :])

            plsc.subcore_barrier()

        do_pass(it_hbm, True, True, u1, first=True)
        do_pass(ut_hbm, True, False, i1)
        do_pass(i1, False, True, u2)
        do_pass(u1, False, False, i2)

    return k1(src2d, dst2d, wcat, ut2, it2, twpad)


def _readout(ut, it, u1, i1, u2, i2, uix1d, iix1d):
    mesh = plsc.VectorSubcoreMesh(core_axis_name="c", subcore_axis_name="s")
    EB = B // (NC * NS)

    @functools.partial(
        pl.kernel,
        out_type=jax.ShapeDtypeStruct((B,), jnp.float32),
        mesh=mesh,
        compiler_params=pltpu.CompilerParams(needs_layout_passes=False,
                                             use_tc_tiling_on_sc=False),
        scratch_types=[
            pltpu.VMEM((EB,), jnp.int32),
            pltpu.VMEM((EB,), jnp.int32),
            pltpu.VMEM((EB,), jnp.int32),
            pltpu.VMEM((EB,), jnp.int32),
            pltpu.VMEM((EB, D), jnp.float32),
            pltpu.VMEM((EB, H), jnp.float32),
            pltpu.VMEM((EB, H), jnp.float32),
            pltpu.VMEM((EB, H), jnp.float32),
            pltpu.VMEM((EB, H), jnp.float32),
            pltpu.VMEM((EB, D), jnp.float32),
            pltpu.VMEM((EB, H), jnp.float32),
            pltpu.VMEM((EB, H), jnp.float32),
            pltpu.VMEM((EB, H), jnp.float32),
            pltpu.VMEM((EB, H), jnp.float32),
            pltpu.VMEM((EB,), jnp.float32),
            pltpu.SemaphoreType.DMA,
        ],
    )
    def k2(ut_hbm, it_hbm, u1_hbm, i1_hbm, u2_hbm, i2_hbm, ux_hbm, ix_hbm,
           out,
           uixb, iixb, ugx, igx,
           ub0, u1p0, u1p1, u2p0, u2p1,
           ib0, i1p0, i1p1, i2p0, i2p1,
           obuf, sem):
        c = lax.axis_index("c")
        s = lax.axis_index("s")
        w = s * NC + c

        pltpu.sync_copy(ux_hbm.at[pl.ds(w * EB, EB)], uixb)
        pltpu.sync_copy(ix_hbm.at[pl.ds(w * EB, EB)], iixb)

        @pl.loop(0, EB // 16)
        def _t(kk):
            sl = pl.ds(kk * 16, 16)
            ugx[sl] = uixb[sl] + NU
            igx[sl] = iixb[sl] + NU

        cps = [
            pltpu.async_copy(ut_hbm.at[uixb], ub0, sem),
            pltpu.async_copy(u1_hbm.at[uixb], u1p0, sem),
            pltpu.async_copy(u1_hbm.at[ugx], u1p1, sem),
            pltpu.async_copy(u2_hbm.at[uixb], u2p0, sem),
            pltpu.async_copy(u2_hbm.at[ugx], u2p1, sem),
            pltpu.async_copy(it_hbm.at[iixb], ib0, sem),
            pltpu.async_copy(i1_hbm.at[iixb], i1p0, sem),
            pltpu.async_copy(i1_hbm.at[igx], i1p1, sem),
            pltpu.async_copy(i2_hbm.at[iixb], i2p0, sem),
            pltpu.async_copy(i2_hbm.at[igx], i2p1, sem),
        ]
        for cp in cps:
            cp.wait()

        @pl.loop(0, EB)
        def _e(ee):
            accv = jnp.zeros((16,), jnp.float32)
            for h, (u1p, u2p, i1p, i2p) in enumerate(
                ((u1p0, u2p0, i1p0, i2p0), (u1p1, u2p1, i1p1, i2p1))):
                for kk in range(2):
                    sl0 = pl.ds(h * 32 + kk * 16, 16)
                    slh = pl.ds(kk * 16, 16)
                    a = ub0[ee, sl0] + u1p[ee, slh] + u2p[ee, slh]
                    b = ib0[ee, sl0] + i1p[ee, slh] + i2p[ee, slh]
                    accv = accv + a * b
            dot = jnp.sum(accv) * (1.0 / 9.0)
            lane0 = lax.iota(jnp.int32, 16) == 0
            plsc.store_scatter(obuf, [jnp.broadcast_to(ee, (16,))],
                               jnp.broadcast_to(dot, (16,)), mask=lane0)

        pltpu.sync_copy(obuf, out.at[pl.ds(w * EB, EB)])

    return k2(ut, it, u1, i1, u2, i2, uix1d, iix1d)


def kernel(user_indices, item_indices, edge_index_t0, weights_t0,
           edge_index_t1, weights_t1, user_table, item_table, type_weights):
    pad = NE_PAD - NE
    pad_idx = (jnp.arange(pad, dtype=jnp.int32) * 37) % NU

    def prep(ei, wts):
        src = jnp.concatenate([ei[0].astype(jnp.int32), pad_idx])
        dst = jnp.concatenate([ei[1].astype(jnp.int32), pad_idx])
        wv = jnp.concatenate([wts.astype(jnp.float32),
                              jnp.zeros((pad,), jnp.float32)])
        return src, dst, wv

    s0, d0, w0 = prep(edge_index_t0, weights_t0)
    s1, d1, w1 = prep(edge_index_t1, weights_t1)
    src2d = jnp.concatenate([s0, s1]).reshape(NROWS, 128)
    dst2d = jnp.concatenate([d0, d1]).reshape(NROWS, 128)
    wcat = jnp.concatenate([w0, w1])
    ut2 = user_table.astype(jnp.float32).reshape(2 * NU, H)
    it2 = item_table.astype(jnp.float32).reshape(2 * NU, H)
    twpad = jnp.concatenate([type_weights.astype(jnp.float32),
                             jnp.full((14,), -jnp.inf, jnp.float32)])
    uix1d = user_indices.astype(jnp.int32)
    iix1d = item_indices.astype(jnp.int32)

    u1, i1, u2, i2 = _propagate(src2d, dst2d, wcat, ut2, it2, twpad)
    return _readout(user_table.astype(jnp.float32),
                    item_table.astype(jnp.float32),
                    u1, i1, u2, i2, uix1d, iix1d)


# speedup vs baseline: 1.0113x; 1.0113x over previous
import functools

import jax
import jax.numpy as jnp
from jax import lax
from jax.experimental import pallas as pl
from jax.experimental.pallas import tpu as pltpu
from jax.experimental.pallas import tpu_sc as plsc

NU = 50000
D = 64
H = 32
NE = 800000
B = 4096
NC = 2
NS = 16
W = 256
RW = W // 128
WIN = 396
EPT = W * WIN
NE_PAD = EPT * 8
NE2 = 2 * NE_PAD
NROWS = NE2 // 128
ACCR = 50048
ZR = 32
DR = 128


def _propagate(src2d, dst2d, wcat, ut2, it2, twpad):
    mesh = plsc.VectorSubcoreMesh(core_axis_name="c", subcore_axis_name="s")
    otype = [jax.ShapeDtypeStruct((2 * NU, H), jnp.float32)] * 4

    ibuf_types = []
    for _ in range(6):
        ibuf_types += [
            pltpu.VMEM((RW, 128), jnp.int32),
            pltpu.VMEM((RW, 128), jnp.int32),
            pltpu.VMEM((W,), jnp.float32),
            pltpu.SemaphoreType.DMA,
        ]

    rbuf_types = []
    for _ in range(3):
        rbuf_types += [
            pltpu.VMEM((W, H), jnp.float32),
            pltpu.SemaphoreType.DMA,
            pltpu.SemaphoreType.DMA,
        ]

    @functools.partial(
        pl.kernel,
        out_type=otype,
        mesh=mesh,
        compiler_params=pltpu.CompilerParams(needs_layout_passes=False,
                                             use_tc_tiling_on_sc=False),
        scratch_types=ibuf_types + rbuf_types + [
            pltpu.VMEM_SHARED((ACCR, H), jnp.float32),
            pltpu.VMEM((16,), jnp.float32),
            pltpu.VMEM((ZR, H), jnp.float32),
        ],
    )
    def k1(src_hbm, dst_hbm, w_hbm, ut_hbm, it_hbm, tw_hbm,
           u1, i1, u2, i2,
           sb0, gb0, wb0, is0, sb1, gb1, wb1, is1,
           sb2, gb2, wb2, is2, sb3, gb3, wb3, is3,
           sb4, gb4, wb4, is4, sb5, gb5, wb5, is5,
           row0, gs0, ss0, row1, gs1, ss1, row2, gs2, ss2,
           acc, twv, zbuf):
        c = lax.axis_index("c")
        s = lax.axis_index("s")
        ins = ((sb0, gb0, wb0, is0), (sb1, gb1, wb1, is1),
               (sb2, gb2, wb2, is2), (sb3, gb3, wb3, is3),
               (sb4, gb4, wb4, is4), (sb5, gb5, wb5, is5))
        rows = ((row0, gs0, ss0), (row1, gs1, ss1), (row2, gs2, ss2))

        pltpu.sync_copy(tw_hbm, twv)
        x = twv[...]
        e = jnp.exp(x - jnp.max(x))
        tw_all = e / jnp.broadcast_to(jnp.sum(e), (16,))
        my_tw = jnp.where(s < 8, tw_all[0], tw_all[1])

        @pl.loop(0, ZR)
        def _zb(r):
            zero16 = jnp.zeros((16,), jnp.float32)
            zbuf[r, pl.ds(0, 16)] = zero16
            zbuf[r, pl.ds(16, 16)] = zero16

        def do_pass(gather_tbl, interleaved, scat_is_src, out_tbl):
            sidx_hbm = src_hbm if scat_is_src else dst_hbm
            gath_hbm = dst_hbm if scat_is_src else src_hbm

            @pl.loop(s, ACCR // ZR, step=NS)
            def _z(m):
                pltpu.sync_copy(zbuf, acc.at[pl.ds(m * ZR, ZR), :])

            def in_copies(nwin, bi):
                sb, gb, wb, isem = ins[bi]
                rowbase = s * (EPT // 128) + nwin * RW
                return (
                    pltpu.make_async_copy(
                        sidx_hbm.at[pl.ds(rowbase, RW), :], sb, isem),
                    pltpu.make_async_copy(
                        gath_hbm.at[pl.ds(rowbase, RW), :], gb, isem),
                    pltpu.make_async_copy(
                        w_hbm.at[pl.ds(rowbase * 128, W)], wb, isem),
                )

            def issue_in(nwin, bi):
                for cp in in_copies(nwin, bi):
                    cp.start()

            def wait_in(nwin, bi):
                for cp in in_copies(nwin, bi):
                    cp.wait()

            def transform(bi):
                _, gb, wb, _ = ins[bi]

                @pl.loop(0, RW)
                def _t(j):
                    for kk in range(8):
                        sl = pl.ds(kk * 16, 16)
                        v = gb[j, sl]
                        if interleaved:
                            gb[j, sl] = v * 2 + c
                        else:
                            gb[j, sl] = v + c * NU

            def gather_copies(bi, ri):
                _, gb, _, _ = ins[bi]
                row, gsem, _ = rows[ri]
                return [
                    pltpu.make_async_copy(
                        gather_tbl.at[gb.at[j]],
                        row.at[pl.ds(j * 128, 128), :], gsem)
                    for j in range(RW)
                ]

            def start_gather(bi, ri):
                for cp in gather_copies(bi, ri):
                    cp.start()

            def wait_gather(bi, ri):
                for cp in gather_copies(bi, ri):
                    cp.wait()

            def multiply(bi, ri):
                _, _, wb, _ = ins[bi]
                row, _, _ = rows[ri]

                @plsc.parallel_loop(0, W // 16)
                def _m(g):
                    w16 = wb[pl.ds(g * 16, 16)] * my_tw
                    for l in range(16):
                        wv = jnp.broadcast_to(w16[l], (16,))
                        ee = g * 16 + l
                        row[ee, pl.ds(0, 16)] = row[ee, pl.ds(0, 16)] * wv
                        row[ee, pl.ds(16, 16)] = row[ee, pl.ds(16, 16)] * wv

            def start_scatter(bi, ri):
                sb, _, _, _ = ins[bi]
                row, _, ssem = rows[ri]
                for j in range(RW):
                    pltpu.async_copy(
                        row.at[pl.ds(j * 128, 128), :],
                        acc.at[sb.at[j]], ssem, add=True)

            def wait_scatter(bi, ri):
                sb, _, _, _ = ins[bi]
                row, _, ssem = rows[ri]
                for j in range(RW):
                    pltpu.make_async_copy(
                        row.at[pl.ds(j * 128, 128), :],
                        acc.at[sb.at[j]], ssem).wait()

            issue_in(0, 0)
            issue_in(1, 1)
            issue_in(2, 2)
            plsc.subcore_barrier()

            NQ = WIN // 6

            @pl.loop(0, NQ)
            def _six(q):
                v0 = 6 * q
                for k in range(6):
                    v = v0 + k
                    ik3 = (k + 3) % 6
                    ik4 = (k + 4) % 6
                    r = k % 3
                    r1 = (k + 1) % 3
                    wait_in(v, k)
                    transform(k)
                    if k <= 2:
                        @pl.when(q > 0)
                        def _ws(ik3=ik3, r=r):
                            wait_scatter(ik3, r)

                        issue_in(v + 3, ik3)
                    else:
                        wait_scatter(ik3, r)

                        @pl.when(q < NQ - 1)
                        def _ii(v=v, ik3=ik3):
                            issue_in(v + 3, ik3)

                    start_gather(k, r)

                    def _tail(ik4=ik4, r1=r1):
                        wait_gather(ik4, r1)
                        multiply(ik4, r1)
                        start_scatter(ik4, r1)

                    if k <= 1:
                        pl.when(q > 0)(_tail)
                    else:
                        _tail()

            wait_gather(4, 1)
            multiply(4, 1)
            start_scatter(4, 1)
            wait_gather(5, 2)
            multiply(5, 2)
            start_scatter(5, 2)
            wait_scatter(3, 0)
            wait_scatter(4, 1)
            wait_scatter(5, 2)

            plsc.subcore_barrier()

            @pl.loop(s, NU // DR, step=NS)
            def _d(m):
                r0 = m * DR
                pltpu.sync_copy(
                    acc.at[pl.ds(r0, DR), :],
                    out_tbl.at[pl.ds(c * NU + r0, DR), :],
                )

            @pl.when(s == (NU // DR) % NS)
            def _tail():
                r0 = (NU // DR) * DR
                pltpu.sync_copy(
                    acc.at[pl.ds(r0, NU - r0), :],
                    out_tbl.at[pl.ds(c * NU + r0, NU - r0), :],
                )

            plsc.subcore_barrier()

        do_pass(it_hbm, True, True, u1)
        do_pass(ut_hbm, True, False, i1)
        do_pass(i1, False, True, u2)
        do_pass(u1, False, False, i2)

    return k1(src2d, dst2d, wcat, ut2, it2, twpad)


def _readout(ut, it, u1, i1, u2, i2, uix1d, iix1d):
    mesh = plsc.VectorSubcoreMesh(core_axis_name="c", subcore_axis_name="s")
    EB = B // (NC * NS)

    @functools.partial(
        pl.kernel,
        out_type=jax.ShapeDtypeStruct((B,), jnp.float32),
        mesh=mesh,
        compiler_params=pltpu.CompilerParams(needs_layout_passes=False,
                                             use_tc_tiling_on_sc=False),
        scratch_types=[
            pltpu.VMEM((EB,), jnp.int32),
            pltpu.VMEM((EB,), jnp.int32),
            pltpu.VMEM((EB,), jnp.int32),
            pltpu.VMEM((EB,), jnp.int32),
            pltpu.VMEM((EB, D), jnp.float32),
            pltpu.VMEM((EB, H), jnp.float32),
            pltpu.VMEM((EB, H), jnp.float32),
            pltpu.VMEM((EB, H), jnp.float32),
            pltpu.VMEM((EB, H), jnp.float32),
            pltpu.VMEM((EB, D), jnp.float32),
            pltpu.VMEM((EB, H), jnp.float32),
            pltpu.VMEM((EB, H), jnp.float32),
            pltpu.VMEM((EB, H), jnp.float32),
            pltpu.VMEM((EB, H), jnp.float32),
            pltpu.VMEM((EB,), jnp.float32),
            pltpu.SemaphoreType.DMA,
        ],
    )
    def k2(ut_hbm, it_hbm, u1_hbm, i1_hbm, u2_hbm, i2_hbm, ux_hbm, ix_hbm,
           out,
           uixb, iixb, ugx, igx,
           ub0, u1p0, u1p1, u2p0, u2p1,
           ib0, i1p0, i1p1, i2p0, i2p1,
           obuf, sem):
        c = lax.axis_index("c")
        s = lax.axis_index("s")
        w = s * NC + c

        pltpu.sync_copy(ux_hbm.at[pl.ds(w * EB, EB)], uixb)
        pltpu.sync_copy(ix_hbm.at[pl.ds(w * EB, EB)], iixb)

        @pl.loop(0, EB // 16)
        def _t(kk):
            sl = pl.ds(kk * 16, 16)
            ugx[sl] = uixb[sl] + NU
            igx[sl] = iixb[sl] + NU

        cps = [
            pltpu.async_copy(ut_hbm.at[uixb], ub0, sem),
            pltpu.async_copy(u1_hbm.at[uixb], u1p0, sem),
            pltpu.async_copy(u1_hbm.at[ugx], u1p1, sem),
            pltpu.async_copy(u2_hbm.at[uixb], u2p0, sem),
            pltpu.async_copy(u2_hbm.at[ugx], u2p1, sem),
            pltpu.async_copy(it_hbm.at[iixb], ib0, sem),
            pltpu.async_copy(i1_hbm.at[iixb], i1p0, sem),
            pltpu.async_copy(i1_hbm.at[igx], i1p1, sem),
            pltpu.async_copy(i2_hbm.at[iixb], i2p0, sem),
            pltpu.async_copy(i2_hbm.at[igx], i2p1, sem),
        ]
        for cp in cps:
            cp.wait()

        @pl.loop(0, EB)
        def _e(ee):
            accv = jnp.zeros((16,), jnp.float32)
            for h, (u1p, u2p, i1p, i2p) in enumerate(
                ((u1p0, u2p0, i1p0, i2p0), (u1p1, u2p1, i1p1, i2p1))):
                for kk in range(2):
                    sl0 = pl.ds(h * 32 + kk * 16, 16)
                    slh = pl.ds(kk * 16, 16)
                    a = ub0[ee, sl0] + u1p[ee, slh] + u2p[ee, slh]
                    b = ib0[ee, sl0] + i1p[ee, slh] + i2p[ee, slh]
                    accv = accv + a * b
            dot = jnp.sum(accv) * (1.0 / 9.0)
            lane0 = lax.iota(jnp.int32, 16) == 0
            plsc.store_scatter(obuf, [jnp.broadcast_to(ee, (16,))],
                               jnp.broadcast_to(dot, (16,)), mask=lane0)

        pltpu.sync_copy(obuf, out.at[pl.ds(w * EB, EB)])

    return k2(ut, it, u1, i1, u2, i2, uix1d, iix1d)


def kernel(user_indices, item_indices, edge_index_t0, weights_t0,
           edge_index_t1, weights_t1, user_table, item_table, type_weights):
    pad = NE_PAD - NE
    pad_idx = (jnp.arange(pad, dtype=jnp.int32) * 37) % NU

    def prep(ei, wts):
        src = jnp.concatenate([ei[0].astype(jnp.int32), pad_idx])
        dst = jnp.concatenate([ei[1].astype(jnp.int32), pad_idx])
        wv = jnp.concatenate([wts.astype(jnp.float32),
                              jnp.zeros((pad,), jnp.float32)])
        return src, dst, wv

    s0, d0, w0 = prep(edge_index_t0, weights_t0)
    s1, d1, w1 = prep(edge_index_t1, weights_t1)
    src2d = jnp.concatenate([s0, s1]).reshape(NROWS, 128)
    dst2d = jnp.concatenate([d0, d1]).reshape(NROWS, 128)
    wcat = jnp.concatenate([w0, w1])
    ut2 = user_table.astype(jnp.float32).reshape(2 * NU, H)
    it2 = item_table.astype(jnp.float32).reshape(2 * NU, H)
    twpad = jnp.concatenate([type_weights.astype(jnp.float32),
                             jnp.full((14,), -jnp.inf, jnp.float32)])
    uix1d = user_indices.astype(jnp.int32)
    iix1d = item_indices.astype(jnp.int32)

    u1, i1, u2, i2 = _propagate(src2d, dst2d, wcat, ut2, it2, twpad)
    return _readout(user_table.astype(jnp.float32),
                    item_table.astype(jnp.float32),
                    u1, i1, u2, i2, uix1d, iix1d)
